# baseline (speedup 1.0000x reference)
import functools

import jax
import jax.numpy as jnp
from jax import lax
from jax.experimental import pallas as pl
from jax.experimental.pallas import tpu as pltpu
from jax.experimental.pallas import tpu_sc as plsc

VOCAB = 1000000
EMBED_DIM = 64
BATCH = 4096
SEQ = 200

NUM_CORES = 2
NUM_SUBCORES = 16
NW = NUM_CORES * NUM_SUBCORES
TOTAL = BATCH * SEQ
PER_W = TOTAL // NW
CHUNK = 128
STEPS = PER_W // CHUNK
NBUF = 3
LOOKAHEAD = 2
PAIRS_W = PER_W // 2

_mesh = plsc.VectorSubcoreMesh(core_axis_name="c", subcore_axis_name="s")


@functools.partial(
    pl.kernel,
    out_type=jax.ShapeDtypeStruct((TOTAL // 2, 128), jnp.float32),
    mesh=_mesh,
    scratch_types=[
        pltpu.VMEM((STEPS, CHUNK), jnp.int32),
        pltpu.VMEM((STEPS, CHUNK), jnp.int32),
        pltpu.VMEM((NBUF, CHUNK, 128), jnp.float32),
        pltpu.VMEM((NBUF, CHUNK // 2, 128), jnp.float32),
        pltpu.SemaphoreType.DMA,
        pltpu.SemaphoreType.DMA,
    ],
    compiler_params=pltpu.CompilerParams(use_tc_tiling_on_sc=False,
                                         needs_layout_passes=False),
)
def _sc_gather(p_hbm, hb_hbm, table_hbm, out_hbm, p_v, hb_v, pair_v, out_v,
               gsem, osem):
    wid = lax.axis_index("s") * NUM_CORES + lax.axis_index("c")
    out0 = wid * PAIRS_W
    pltpu.sync_copy(p_hbm.at[wid], p_v)
    pltpu.sync_copy(hb_hbm.at[wid], hb_v)

    lane = lax.iota(jnp.int32, 16)
    pcol = (lane & 1) << 6

    def fire_gather(s, buf):
        pltpu.async_copy(table_hbm.at[p_v.at[s]], pair_v.at[buf], gsem)

    def wait_gather(buf):
        pltpu.make_async_copy(
            table_hbm.at[pl.ds(0, CHUNK)], pair_v.at[buf], gsem).wait()

    def wait_store(buf):
        pltpu.make_async_copy(
            out_v.at[buf], out_hbm.at[pl.ds(0, CHUNK // 2)], osem).wait()

    for s0 in range(LOOKAHEAD):
        fire_gather(s0, s0)

    def superstep(g, carry):
        for sub in range(NBUF):
            s = g * NBUF + sub

            @pl.when(s + LOOKAHEAD < STEPS)
            def _():
                fire_gather(s + LOOKAHEAD, (sub + LOOKAHEAD) % NBUF)

            @pl.when(s < STEPS)
            def _():
                wait_gather(sub)

                @pl.when(s >= NBUF)
                def _():
                    wait_store(sub)

                pv = pair_v.at[sub]
                ov = out_v.at[sub]
                blocks = []
                for blk in range(CHUNK // 16):
                    src_row = blk * 16 + lane
                    dst_row = blk * 8 + (lane >> 1)
                    hb16 = hb_v[s, pl.ds(blk * 16, 16)]
                    blocks.append((src_row, dst_row, hb16))

                def sel(k, c):
                    kb = jnp.full((16,), 0, jnp.int32) + k
                    dcol = pcol + kb
                    for src_row, dst_row, hb16 in blocks:
                        v = plsc.load_gather(pv, [src_row, hb16 + kb])
                        plsc.store_scatter(ov, [dst_row, dcol], v)
                    return c

                lax.fori_loop(0, EMBED_DIM, sel, 0)
                pltpu.async_copy(
                    ov, out_hbm.at[pl.ds(out0 + s * (CHUNK // 2), CHUNK // 2)],
                    osem)
        return carry

    lax.fori_loop(0, (STEPS + NBUF - 1) // NBUF, superstep, 0)
    for _ in range(NBUF):
        wait_store(0)


def kernel(x, embedding_table):
    xf = x.reshape(-1).astype(jnp.int32)
    p = (xf >> 1).reshape(NW, STEPS, CHUNK)
    hb = ((xf & 1) << 6).reshape(NW, STEPS, CHUNK)
    table2 = embedding_table.reshape(VOCAB // 2, 128)
    out = _sc_gather(p, hb, table2)
    return out.reshape(BATCH, SEQ, EMBED_DIM)


# speedup vs baseline: 1.6213x; 1.6213x over previous
import functools

import jax
import jax.numpy as jnp
from jax import lax
from jax.experimental import pallas as pl
from jax.experimental.pallas import tpu as pltpu
from jax.experimental.pallas import tpu_sc as plsc

VOCAB = 1000000
EMBED_DIM = 64
BATCH = 4096
SEQ = 200

NUM_CORES = 2
NUM_SUBCORES = 16
NW = NUM_CORES * NUM_SUBCORES
TOTAL = BATCH * SEQ
PER_W = TOTAL // NW
CHUNK = 128
STEPS = PER_W // CHUNK
NBUF = 3
LOOKAHEAD = 2
PAIRS_W = PER_W // 2

_mesh = plsc.VectorSubcoreMesh(core_axis_name="c", subcore_axis_name="s")


@functools.partial(
    pl.kernel,
    out_type=jax.ShapeDtypeStruct((TOTAL // 2, 128), jnp.float32),
    mesh=_mesh,
    scratch_types=[
        pltpu.VMEM((STEPS, CHUNK), jnp.int32),
        pltpu.VMEM((STEPS, CHUNK), jnp.int32),
        pltpu.VMEM((NBUF, CHUNK, 128), jnp.float32),
        pltpu.VMEM((NBUF, CHUNK // 2, 128), jnp.float32),
        pltpu.SemaphoreType.DMA,
        pltpu.SemaphoreType.DMA,
    ],
    compiler_params=pltpu.CompilerParams(use_tc_tiling_on_sc=True,
                                         needs_layout_passes=False),
)
def _sc_gather(p_hbm, hb_hbm, table_hbm, out_hbm, p_v, hb_v, pair_v, out_v,
               gsem, osem):
    wid = lax.axis_index("s") * NUM_CORES + lax.axis_index("c")
    out0 = wid * PAIRS_W
    pltpu.sync_copy(p_hbm.at[wid], p_v)
    pltpu.sync_copy(hb_hbm.at[wid], hb_v)

    lane = lax.iota(jnp.int32, 16)
    pcol = (lane & 1) << 6

    def fire_gather(s, buf):
        pltpu.async_copy(table_hbm.at[p_v.at[s]], pair_v.at[buf], gsem)

    def wait_gather(buf):
        pltpu.make_async_copy(
            table_hbm.at[pl.ds(0, CHUNK)], pair_v.at[buf], gsem).wait()

    def wait_store(buf):
        pltpu.make_async_copy(
            out_v.at[buf], out_hbm.at[pl.ds(0, CHUNK // 2)], osem).wait()

    for s0 in range(LOOKAHEAD):
        fire_gather(s0, s0)

    def superstep(g, carry):
        for sub in range(NBUF):
            s = g * NBUF + sub

            @pl.when(s + LOOKAHEAD < STEPS)
            def _():
                fire_gather(s + LOOKAHEAD, (sub + LOOKAHEAD) % NBUF)

            @pl.when(s < STEPS)
            def _():
                wait_gather(sub)

                @pl.when(s >= NBUF)
                def _():
                    wait_store(sub)

                pv = pair_v.at[sub]
                ov = out_v.at[sub]
                blocks = []
                for blk in range(CHUNK // 16):
                    src_row = blk * 16 + lane
                    dst_row = blk * 8 + (lane >> 1)
                    hb16 = hb_v[s, pl.ds(blk * 16, 16)]
                    blocks.append((src_row, dst_row, hb16))

                @plsc.parallel_loop(0, EMBED_DIM, unroll=2)
                def _(k):
                    kb = jnp.full((16,), 0, jnp.int32) + k
                    dcol = pcol + kb
                    for src_row, dst_row, hb16 in blocks:
                        v = plsc.load_gather(pv, [src_row, hb16 + kb])
                        plsc.store_scatter(ov, [dst_row, dcol], v)
                pltpu.async_copy(
                    ov, out_hbm.at[pl.ds(out0 + s * (CHUNK // 2), CHUNK // 2)],
                    osem)
        return carry

    lax.fori_loop(0, (STEPS + NBUF - 1) // NBUF, superstep, 0)
    for _ in range(NBUF):
        wait_store(0)


def kernel(x, embedding_table):
    xf = x.reshape(-1).astype(jnp.int32)
    p = (xf >> 1).reshape(NW, STEPS, CHUNK)
    hb = ((xf & 1) << 6).reshape(NW, STEPS, CHUNK)
    table2 = embedding_table.reshape(VOCAB // 2, 128)
    out = _sc_gather(p, hb, table2)
    return out.reshape(BATCH, SEQ, EMBED_DIM)


# speedup vs baseline: 2.2965x; 1.4165x over previous
import functools

import jax
import jax.numpy as jnp
from jax import lax
from jax.experimental import pallas as pl
from jax.experimental.pallas import tpu as pltpu
from jax.experimental.pallas import tpu_sc as plsc

VOCAB = 1000000
EMBED_DIM = 64
BATCH = 4096
SEQ = 200

NUM_CORES = 2
NUM_SUBCORES = 16
NW = NUM_CORES * NUM_SUBCORES
TOTAL = BATCH * SEQ
PER_W = TOTAL // NW
CHUNK = 128
STEPS = PER_W // CHUNK
NBUF = 2
LOOKAHEAD = 1

_mesh = plsc.VectorSubcoreMesh(core_axis_name="c", subcore_axis_name="s")


@functools.partial(
    pl.kernel,
    out_type=jax.ShapeDtypeStruct((TOTAL, EMBED_DIM), jnp.float32),
    mesh=_mesh,
    scratch_types=[
        pltpu.VMEM((STEPS, CHUNK), jnp.int32),
        pltpu.VMEM((STEPS, CHUNK), jnp.int32),
        pltpu.VMEM((NBUF, CHUNK, 128), jnp.float32),
        pltpu.VMEM((NBUF, CHUNK, EMBED_DIM), jnp.float32),
        pltpu.SemaphoreType.DMA,
        pltpu.SemaphoreType.DMA,
    ],
    compiler_params=pltpu.CompilerParams(use_tc_tiling_on_sc=True,
                                         needs_layout_passes=False),
)
def _sc_gather(p_hbm, hb_hbm, table_hbm, out_hbm, p_v, hb_v, pair_v, out_v,
               gsem, osem):
    wid = lax.axis_index("s") * NUM_CORES + lax.axis_index("c")
    base = wid * PER_W
    pltpu.sync_copy(p_hbm.at[wid], p_v)
    pltpu.sync_copy(hb_hbm.at[wid], hb_v)

    lane = lax.iota(jnp.int32, 16)

    def fire_gather(s, buf):
        pltpu.async_copy(table_hbm.at[p_v.at[s]], pair_v.at[buf], gsem)

    def wait_gather(buf):
        pltpu.make_async_copy(
            table_hbm.at[pl.ds(0, CHUNK)], pair_v.at[buf], gsem).wait()

    def wait_store(buf):
        pltpu.make_async_copy(
            out_v.at[buf], out_hbm.at[pl.ds(0, CHUNK)], osem).wait()

    for s0 in range(LOOKAHEAD):
        fire_gather(s0, s0)

    def superstep(g, carry):
        for sub in range(NBUF):
            s = g * NBUF + sub

            @pl.when(s + LOOKAHEAD < STEPS)
            def _():
                fire_gather(s + LOOKAHEAD, (sub + LOOKAHEAD) % NBUF)

            @pl.when(s < STEPS)
            def _():
                wait_gather(sub)

                @pl.when(s >= NBUF)
                def _():
                    wait_store(sub)

                pv = pair_v.at[sub]
                ov = out_v.at[sub]
                blocks = []
                for blk in range(CHUNK // 16):
                    row16 = blk * 16 + lane
                    hb16 = hb_v[s, pl.ds(blk * 16, 16)]
                    blocks.append((row16, hb16))

                @plsc.parallel_loop(0, EMBED_DIM, unroll=4)
                def _(k):
                    kb = jnp.full((16,), 0, jnp.int32) + k
                    for row16, hb16 in blocks:
                        v = plsc.load_gather(pv, [row16, hb16 + kb])
                        plsc.store_scatter(ov, [row16, kb], v)

                pltpu.async_copy(
                    ov, out_hbm.at[pl.ds(base + s * CHUNK, CHUNK)], osem)
        return carry

    lax.fori_loop(0, (STEPS + NBUF - 1) // NBUF, superstep, 0)
    for _ in range(NBUF):
        wait_store(0)


def kernel(x, embedding_table):
    xf = x.reshape(-1).astype(jnp.int32)
    p = (xf >> 1).reshape(NW, STEPS, CHUNK)
    hb = ((xf & 1) << 6).reshape(NW, STEPS, CHUNK)
    table2 = embedding_table.reshape(VOCAB // 2, 128)
    out = _sc_gather(p, hb, table2)
    return out.reshape(BATCH, SEQ, EMBED_DIM)


# speedup vs baseline: 2.3882x; 1.0399x over previous
import functools

import jax
import jax.numpy as jnp
from jax import lax
from jax.experimental import pallas as pl
from jax.experimental.pallas import tpu as pltpu
from jax.experimental.pallas import tpu_sc as plsc

VOCAB = 1000000
EMBED_DIM = 64
BATCH = 4096
SEQ = 200

NUM_CORES = 2
NUM_SUBCORES = 16
NW = NUM_CORES * NUM_SUBCORES
TOTAL = BATCH * SEQ
PER_W = TOTAL // NW
CHUNK = 128
STEPS = PER_W // CHUNK
NBUF = 3
LOOKAHEAD = 2

_mesh = plsc.VectorSubcoreMesh(core_axis_name="c", subcore_axis_name="s")


@functools.partial(
    pl.kernel,
    out_type=jax.ShapeDtypeStruct((TOTAL, EMBED_DIM), jnp.float32),
    mesh=_mesh,
    scratch_types=[
        pltpu.VMEM((STEPS, CHUNK), jnp.int32),
        pltpu.VMEM((NBUF, CHUNK, EMBED_DIM), jnp.float32),
        pltpu.SemaphoreType.DMA,
        pltpu.SemaphoreType.DMA,
    ],
    compiler_params=pltpu.CompilerParams(use_tc_tiling_on_sc=False),
)
def _sc_gather(idx_hbm, table_hbm, out_hbm, idx_v, rows_v, gsem, osem):
    wid = lax.axis_index("s") * NUM_CORES + lax.axis_index("c")
    base = wid * PER_W
    pltpu.sync_copy(idx_hbm.at[wid], idx_v)

    def fire_gather(s, buf):
        pltpu.async_copy(table_hbm.at[idx_v.at[s]], rows_v.at[buf], gsem)

    def wait_gather(buf):
        pltpu.make_async_copy(
            table_hbm.at[pl.ds(0, CHUNK)], rows_v.at[buf], gsem).wait()

    def wait_store(buf):
        pltpu.make_async_copy(
            rows_v.at[buf], out_hbm.at[pl.ds(0, CHUNK)], osem).wait()

    for s0 in range(LOOKAHEAD):
        fire_gather(s0, s0)

    def superstep(g, carry):
        for sub in range(NBUF):
            s = g * NBUF + sub

            @pl.when(s + LOOKAHEAD < STEPS)
            def _():
                fire_gather(s + LOOKAHEAD, (sub + LOOKAHEAD) % NBUF)

            @pl.when(s < STEPS)
            def _():
                wait_gather(sub)

                @pl.when(s >= NBUF)
                def _():
                    wait_store(sub)

                pltpu.async_copy(
                    rows_v.at[sub],
                    out_hbm.at[pl.ds(base + s * CHUNK, CHUNK)], osem)
        return carry

    lax.fori_loop(0, (STEPS + NBUF - 1) // NBUF, superstep, 0)
    for _ in range(NBUF):
        wait_store(0)


def kernel(x, embedding_table):
    idx = x.reshape(NW, STEPS, CHUNK).astype(jnp.int32)
    out = _sc_gather(idx, embedding_table)
    return out.reshape(BATCH, SEQ, EMBED_DIM)
